# Initial kernel scaffold; baseline (speedup 1.0000x reference)
import jax
import jax.numpy as jnp
from jax.experimental import pallas as pl


def kernel(edge_index, x, W1_rel, b1_rel, W1_root, W2_rel, b2_rel, W2_root, W3_rel, b3_rel, W3_root):
    raise NotImplementedError("write your pallas kernel here")


# speedup vs baseline: 4.9145x; 4.9145x over previous
import functools

import jax
import jax.numpy as jnp
from jax import lax
from jax.experimental import pallas as pl
from jax.experimental.pallas import tpu as pltpu
from jax.experimental.pallas import tpu_sc as plsc

N = 10000
E = 160000
D = 256
NP = 10240
NPF = NP * 4
NR = NP // 32
NTILES = 16
NCORES = 2
CS = 512
CH = 40
EPF = NCORES * NTILES * CH * CS


def _tc1_body(x_ref, wr_ref, wo_ref, p_ref, q_ref):
    xb = x_ref[...]
    p_ref[...] = jnp.dot(xb, wr_ref[...], preferred_element_type=jnp.float32)
    q_ref[...] = jnp.dot(xb, wo_ref[...], preferred_element_type=jnp.float32)


def _tc1(x_r, wr_blk, wo_blk):
    return pl.pallas_call(
        _tc1_body,
        out_shape=(
            jax.ShapeDtypeStruct((NR, 128), jnp.float32),
            jax.ShapeDtypeStruct((NR, 128), jnp.float32),
        ),
    )(x_r, wr_blk, wo_blk)


def _tcmid_body(parts_ref, q_ref, b_ref, wr_ref, wo_ref, pn_ref, qn_ref):
    h = jnp.tanh(parts_ref[0] + parts_ref[1] + q_ref[...] + b_ref[...])
    pn_ref[...] = jnp.dot(h, wr_ref[...], preferred_element_type=jnp.float32)
    qn_ref[...] = jnp.dot(h, wo_ref[...], preferred_element_type=jnp.float32)


def _tcmid(parts, q, b_row, wr_blk, wo_blk):
    return pl.pallas_call(
        _tcmid_body,
        out_shape=(
            jax.ShapeDtypeStruct((NR, 128), jnp.float32),
            jax.ShapeDtypeStruct((NR, 128), jnp.float32),
        ),
    )(parts, q, b_row, wr_blk, wo_blk)


def _tcfin_body(parts_ref, q_ref, b_ref, o_ref):
    o_ref[...] = jnp.tanh(parts_ref[0] + parts_ref[1] + q_ref[...] + b_ref[...])


def _tcfin(parts, q, b_row):
    return pl.pallas_call(
        _tcfin_body,
        out_shape=jax.ShapeDtypeStruct((NR, 128), jnp.float32),
    )(parts, q, b_row)


@functools.partial(
    pl.kernel,
    out_type=jax.ShapeDtypeStruct((NCORES, NPF), jnp.float32),
    mesh=plsc.VectorSubcoreMesh(core_axis_name="c", subcore_axis_name="s"),
    scratch_types=[
        pltpu.VMEM_SHARED((NPF,), jnp.float32),
        pltpu.VMEM_SHARED((NPF,), jnp.float32),
        pltpu.VMEM((CS,), jnp.int32),
        pltpu.VMEM((CS,), jnp.int32),
        pltpu.VMEM((CS,), jnp.float32),
        pltpu.SemaphoreType.DMA,
    ],
)
def _sc_edge_agg(p_hbm, src_hbm, dst_hbm, z_hbm, out_hbm,
                 p_sp, agg_sp, sidx, didx, rows, sem):
    c = lax.axis_index("c")
    s = lax.axis_index("s")
    rp = NPF // NTILES
    sl = pl.ds(s * rp, rp)
    pltpu.sync_copy(p_hbm.at[sl], p_sp.at[sl])
    pltpu.sync_copy(z_hbm.at[sl], agg_sp.at[sl])
    plsc.subcore_barrier()

    def chunk(i, carry):
        pltpu.sync_copy(src_hbm.at[c, s, i], sidx)
        pltpu.sync_copy(dst_hbm.at[c, s, i], didx)
        pltpu.async_copy(p_sp.at[sidx], rows, sem).wait()
        pltpu.sync_copy(rows, agg_sp.at[didx], add=True)
        return carry
    lax.fori_loop(0, CH, chunk, 0)

    plsc.subcore_barrier()
    pltpu.sync_copy(agg_sp.at[sl], out_hbm.at[c].at[sl])


def _expand(idx):
    e4 = (idx[:, None] * 4 + jnp.arange(4, dtype=jnp.int32)).reshape(-1)
    npad = EPF - 4 * E
    pad = (4 * N + (jnp.arange(npad, dtype=jnp.int32) % (NPF - 4 * N)))
    return jnp.concatenate([e4, pad]).reshape(NCORES, NTILES, CH, CS)


def _blk(w, pad_to=4):
    wp = jnp.pad(w, ((0, 4 - w.shape[0]), (0, 4 - w.shape[1])))
    return jnp.kron(jnp.eye(32, dtype=jnp.float32), wp)


def kernel(edge_index, x, W1_rel, b1_rel, W1_root,
           W2_rel, b2_rel, W2_root, W3_rel, b3_rel, W3_root):
    src4 = _expand(edge_index[0])
    dst4 = _expand(edge_index[1])
    x_r = jnp.pad(x, ((0, NP - N), (0, 0))).reshape(NR, 32 * D)
    zeros = jnp.zeros((NPF,), jnp.float32)

    w1r_blk = jnp.kron(jnp.eye(32, dtype=jnp.float32), W1_rel)
    w1o_blk = jnp.kron(jnp.eye(32, dtype=jnp.float32), W1_root)
    p1, q1 = _tc1(x_r, w1r_blk, w1o_blk)
    parts1 = _sc_edge_agg(p1.reshape(NPF), src4, dst4, zeros)

    b1t = jnp.tile(b1_rel, 32).reshape(1, 128)
    p2, q2 = _tcmid(parts1.reshape(NCORES, NR, 128), q1, b1t,
                    _blk(W2_rel), _blk(W2_root))
    parts2 = _sc_edge_agg(p2.reshape(NPF), src4, dst4, zeros)

    b2t = jnp.tile(b2_rel, 32).reshape(1, 128)
    p3, q3 = _tcmid(parts2.reshape(NCORES, NR, 128), q2, b2t,
                    _blk(W3_rel), _blk(W3_root))
    parts3 = _sc_edge_agg(p3.reshape(NPF), src4, dst4, zeros)

    b3t = jnp.tile(jnp.pad(b3_rel, (0, 2)), 32).reshape(1, 128)
    out = _tcfin(parts3.reshape(NCORES, NR, 128), q3, b3t)
    return out.reshape(NP, 4)[:N, :2]


# speedup vs baseline: 5.7027x; 1.1604x over previous
import functools

import jax
import jax.numpy as jnp
from jax import lax
from jax.experimental import pallas as pl
from jax.experimental.pallas import tpu as pltpu
from jax.experimental.pallas import tpu_sc as plsc

N = 10000
E = 160000
D = 256
NP = 10240
NPF = NP * 4
NR = NP // 32
NTILES = 16
NCORES = 2
CS = 512
CH = 40
EPF = NCORES * NTILES * CH * CS


def _tc1_body(x_ref, wr_ref, wo_ref, p_ref, q_ref):
    xb = x_ref[...]
    p_ref[...] = jnp.dot(xb, wr_ref[...], preferred_element_type=jnp.float32)
    q_ref[...] = jnp.dot(xb, wo_ref[...], preferred_element_type=jnp.float32)


def _tc1(x_r, wr_blk, wo_blk):
    return pl.pallas_call(
        _tc1_body,
        out_shape=(
            jax.ShapeDtypeStruct((NR, 128), jnp.float32),
            jax.ShapeDtypeStruct((NR, 128), jnp.float32),
        ),
    )(x_r, wr_blk, wo_blk)


def _tcmid_body(parts_ref, q_ref, b_ref, wr_ref, wo_ref, pn_ref, qn_ref):
    h = jnp.tanh(parts_ref[0] + parts_ref[1] + q_ref[...] + b_ref[...])
    pn_ref[...] = jnp.dot(h, wr_ref[...], preferred_element_type=jnp.float32)
    qn_ref[...] = jnp.dot(h, wo_ref[...], preferred_element_type=jnp.float32)


def _tcmid(parts, q, b_row, wr_blk, wo_blk):
    return pl.pallas_call(
        _tcmid_body,
        out_shape=(
            jax.ShapeDtypeStruct((NR, 128), jnp.float32),
            jax.ShapeDtypeStruct((NR, 128), jnp.float32),
        ),
    )(parts, q, b_row, wr_blk, wo_blk)


def _tcfin_body(parts_ref, q_ref, b_ref, o_ref):
    o_ref[...] = jnp.tanh(parts_ref[0] + parts_ref[1] + q_ref[...] + b_ref[...])


def _tcfin(parts, q, b_row):
    return pl.pallas_call(
        _tcfin_body,
        out_shape=jax.ShapeDtypeStruct((NR, 128), jnp.float32),
    )(parts, q, b_row)


@functools.partial(
    pl.kernel,
    out_type=jax.ShapeDtypeStruct((NCORES, NPF), jnp.float32),
    mesh=plsc.VectorSubcoreMesh(core_axis_name="c", subcore_axis_name="s"),
    scratch_types=[
        pltpu.VMEM_SHARED((NPF,), jnp.float32),
        pltpu.VMEM_SHARED((NPF,), jnp.float32),
        pltpu.VMEM_SHARED((NTILES, 2, CH, CS), jnp.int32),
        pltpu.VMEM((CS,), jnp.int32),
        pltpu.VMEM((CS,), jnp.int32),
        pltpu.VMEM((CS,), jnp.float32),
        pltpu.SemaphoreType.DMA,
    ],
)
def _sc_edge_agg(p_hbm, idx_hbm, z_hbm, out_hbm,
                 p_sp, agg_sp, idx_sp, sidx, didx, rows, sem):
    c = lax.axis_index("c")
    s = lax.axis_index("s")
    rp = NPF // NTILES
    sl = pl.ds(s * rp, rp)
    pltpu.sync_copy(p_hbm.at[sl], p_sp.at[sl])
    pltpu.sync_copy(z_hbm.at[sl], agg_sp.at[sl])
    pltpu.sync_copy(idx_hbm.at[c, s], idx_sp.at[s])
    plsc.subcore_barrier()

    def chunk(i, carry):
        pltpu.sync_copy(idx_sp.at[s, 0, i], sidx)
        pltpu.sync_copy(idx_sp.at[s, 1, i], didx)
        pltpu.async_copy(p_sp.at[sidx], rows, sem).wait()
        pltpu.sync_copy(rows, agg_sp.at[didx], add=True)
        return carry
    lax.fori_loop(0, CH, chunk, 0)

    plsc.subcore_barrier()
    pltpu.sync_copy(agg_sp.at[sl], out_hbm.at[c].at[sl])


def _expand(idx):
    e4 = (idx[:, None] * 4 + jnp.arange(4, dtype=jnp.int32)).reshape(-1)
    npad = EPF - 4 * E
    pad = (4 * N + (jnp.arange(npad, dtype=jnp.int32) % (NPF - 4 * N)))
    return jnp.concatenate([e4, pad]).reshape(NCORES, NTILES, CH, CS)


def _blk(w, pad_to=4):
    wp = jnp.pad(w, ((0, 4 - w.shape[0]), (0, 4 - w.shape[1])))
    return jnp.kron(jnp.eye(32, dtype=jnp.float32), wp)


def kernel(edge_index, x, W1_rel, b1_rel, W1_root,
           W2_rel, b2_rel, W2_root, W3_rel, b3_rel, W3_root):
    idx4 = jnp.stack([_expand(edge_index[0]), _expand(edge_index[1])], axis=2)
    x_r = jnp.pad(x, ((0, NP - N), (0, 0))).reshape(NR, 32 * D)
    zeros = jnp.zeros((NPF,), jnp.float32)

    w1r_blk = jnp.kron(jnp.eye(32, dtype=jnp.float32), W1_rel)
    w1o_blk = jnp.kron(jnp.eye(32, dtype=jnp.float32), W1_root)
    p1, q1 = _tc1(x_r, w1r_blk, w1o_blk)
    parts1 = _sc_edge_agg(p1.reshape(NPF), idx4, zeros)

    b1t = jnp.tile(b1_rel, 32).reshape(1, 128)
    p2, q2 = _tcmid(parts1.reshape(NCORES, NR, 128), q1, b1t,
                    _blk(W2_rel), _blk(W2_root))
    parts2 = _sc_edge_agg(p2.reshape(NPF), idx4, zeros)

    b2t = jnp.tile(b2_rel, 32).reshape(1, 128)
    p3, q3 = _tcmid(parts2.reshape(NCORES, NR, 128), q2, b2t,
                    _blk(W3_rel), _blk(W3_root))
    parts3 = _sc_edge_agg(p3.reshape(NPF), idx4, zeros)

    b3t = jnp.tile(jnp.pad(b3_rel, (0, 2)), 32).reshape(1, 128)
    out = _tcfin(parts3.reshape(NCORES, NR, 128), q3, b3t)
    return out.reshape(NP, 4)[:N, :2]


# speedup vs baseline: 6.1057x; 1.0707x over previous
import functools

import jax
import jax.numpy as jnp
from jax import lax
from jax.experimental import pallas as pl
from jax.experimental.pallas import tpu as pltpu
from jax.experimental.pallas import tpu_sc as plsc

N = 10000
E = 160000
D = 256
NP = 10240
NPF = NP * 4
NR = NP // 32
NTILES = 16
NCORES = 2
CS = 2560
CH = 8
EPF = NCORES * NTILES * CH * CS


def _tc1_body(x_ref, wr_ref, wo_ref, p_ref, q_ref):
    xb = x_ref[...]
    p_ref[...] = jnp.dot(xb, wr_ref[...], preferred_element_type=jnp.float32)
    q_ref[...] = jnp.dot(xb, wo_ref[...], preferred_element_type=jnp.float32)


def _tc1(x_r, wr_blk, wo_blk):
    return pl.pallas_call(
        _tc1_body,
        out_shape=(
            jax.ShapeDtypeStruct((NR, 128), jnp.float32),
            jax.ShapeDtypeStruct((NR, 128), jnp.float32),
        ),
    )(x_r, wr_blk, wo_blk)


def _tcmid_body(parts_ref, q_ref, b_ref, wr_ref, wo_ref, pn_ref, qn_ref):
    h = jnp.tanh(parts_ref[0] + parts_ref[1] + q_ref[...] + b_ref[...])
    pn_ref[...] = jnp.dot(h, wr_ref[...], preferred_element_type=jnp.float32)
    qn_ref[...] = jnp.dot(h, wo_ref[...], preferred_element_type=jnp.float32)


def _tcmid(parts, q, b_row, wr_blk, wo_blk):
    return pl.pallas_call(
        _tcmid_body,
        out_shape=(
            jax.ShapeDtypeStruct((NR, 128), jnp.float32),
            jax.ShapeDtypeStruct((NR, 128), jnp.float32),
        ),
    )(parts, q, b_row, wr_blk, wo_blk)


def _tcfin_body(parts_ref, q_ref, b_ref, o_ref):
    o_ref[...] = jnp.tanh(parts_ref[0] + parts_ref[1] + q_ref[...] + b_ref[...])


def _tcfin(parts, q, b_row):
    return pl.pallas_call(
        _tcfin_body,
        out_shape=jax.ShapeDtypeStruct((NR, 128), jnp.float32),
    )(parts, q, b_row)


@functools.partial(
    pl.kernel,
    out_type=jax.ShapeDtypeStruct((NCORES, NPF), jnp.float32),
    mesh=plsc.VectorSubcoreMesh(core_axis_name="c", subcore_axis_name="s"),
    scratch_types=[
        pltpu.VMEM_SHARED((NPF,), jnp.float32),
        pltpu.VMEM_SHARED((NPF,), jnp.float32),
        pltpu.VMEM_SHARED((NTILES, 2, CH, CS), jnp.int32),
        pltpu.VMEM((CS,), jnp.int32),
        pltpu.VMEM((CS,), jnp.int32),
        pltpu.VMEM((CS,), jnp.float32),
        pltpu.SemaphoreType.DMA,
    ],
)
def _sc_edge_agg(p_hbm, idx_hbm, z_hbm, out_hbm,
                 p_sp, agg_sp, idx_sp, sidx, didx, rows, sem):
    c = lax.axis_index("c")
    s = lax.axis_index("s")
    rp = NPF // NTILES
    sl = pl.ds(s * rp, rp)
    pltpu.sync_copy(p_hbm.at[sl], p_sp.at[sl])
    pltpu.sync_copy(z_hbm.at[sl], agg_sp.at[sl])
    pltpu.sync_copy(idx_hbm.at[c, s], idx_sp.at[s])
    plsc.subcore_barrier()

    def chunk(i, carry):
        pltpu.sync_copy(idx_sp.at[s, 0, i], sidx)
        pltpu.sync_copy(idx_sp.at[s, 1, i], didx)
        pltpu.async_copy(p_sp.at[sidx], rows, sem).wait()
        pltpu.sync_copy(rows, agg_sp.at[didx], add=True)
        return carry
    lax.fori_loop(0, CH, chunk, 0)

    plsc.subcore_barrier()
    pltpu.sync_copy(agg_sp.at[sl], out_hbm.at[c].at[sl])


def _expand(idx):
    e4 = (idx[:, None] * 4 + jnp.arange(4, dtype=jnp.int32)).reshape(-1)
    npad = EPF - 4 * E
    pad = (4 * N + (jnp.arange(npad, dtype=jnp.int32) % (NPF - 4 * N)))
    return jnp.concatenate([e4, pad]).reshape(NCORES, NTILES, CH, CS)


def _blk(w, pad_to=4):
    wp = jnp.pad(w, ((0, 4 - w.shape[0]), (0, 4 - w.shape[1])))
    return jnp.kron(jnp.eye(32, dtype=jnp.float32), wp)


def kernel(edge_index, x, W1_rel, b1_rel, W1_root,
           W2_rel, b2_rel, W2_root, W3_rel, b3_rel, W3_root):
    idx4 = jnp.stack([_expand(edge_index[0]), _expand(edge_index[1])], axis=2)
    x_r = jnp.pad(x, ((0, NP - N), (0, 0))).reshape(NR, 32 * D)
    zeros = jnp.zeros((NPF,), jnp.float32)

    w1r_blk = jnp.kron(jnp.eye(32, dtype=jnp.float32), W1_rel)
    w1o_blk = jnp.kron(jnp.eye(32, dtype=jnp.float32), W1_root)
    p1, q1 = _tc1(x_r, w1r_blk, w1o_blk)
    parts1 = _sc_edge_agg(p1.reshape(NPF), idx4, zeros)

    b1t = jnp.tile(b1_rel, 32).reshape(1, 128)
    p2, q2 = _tcmid(parts1.reshape(NCORES, NR, 128), q1, b1t,
                    _blk(W2_rel), _blk(W2_root))
    parts2 = _sc_edge_agg(p2.reshape(NPF), idx4, zeros)

    b2t = jnp.tile(b2_rel, 32).reshape(1, 128)
    p3, q3 = _tcmid(parts2.reshape(NCORES, NR, 128), q2, b2t,
                    _blk(W3_rel), _blk(W3_root))
    parts3 = _sc_edge_agg(p3.reshape(NPF), idx4, zeros)

    b3t = jnp.tile(jnp.pad(b3_rel, (0, 2)), 32).reshape(1, 128)
    out = _tcfin(parts3.reshape(NCORES, NR, 128), q3, b3t)
    return out.reshape(NP, 4)[:N, :2]


# speedup vs baseline: 6.7422x; 1.1042x over previous
import functools

import jax
import jax.numpy as jnp
from jax import lax
from jax.experimental import pallas as pl
from jax.experimental.pallas import tpu as pltpu
from jax.experimental.pallas import tpu_sc as plsc

N = 10000
E = 160000
D = 256
NP = 10240
NPF = NP * 4
NR = NP // 32
NTILES = 16
NCORES = 2
CS = 20480
CH = 1
EPF = NCORES * NTILES * CH * CS


def _tc1_body(x_ref, wr_ref, wo_ref, p_ref, q_ref):
    xb = x_ref[...]
    p_ref[...] = jnp.dot(xb, wr_ref[...], preferred_element_type=jnp.float32)
    q_ref[...] = jnp.dot(xb, wo_ref[...], preferred_element_type=jnp.float32)


def _tc1(x_r, wr_blk, wo_blk):
    return pl.pallas_call(
        _tc1_body,
        out_shape=(
            jax.ShapeDtypeStruct((NR, 128), jnp.float32),
            jax.ShapeDtypeStruct((NR, 128), jnp.float32),
        ),
    )(x_r, wr_blk, wo_blk)


def _tcmid_body(parts_ref, q_ref, b_ref, wr_ref, wo_ref, pn_ref, qn_ref):
    h = jnp.tanh(parts_ref[0] + parts_ref[1] + q_ref[...] + b_ref[...])
    pn_ref[...] = jnp.dot(h, wr_ref[...], preferred_element_type=jnp.float32)
    qn_ref[...] = jnp.dot(h, wo_ref[...], preferred_element_type=jnp.float32)


def _tcmid(parts, q, b_row, wr_blk, wo_blk):
    return pl.pallas_call(
        _tcmid_body,
        out_shape=(
            jax.ShapeDtypeStruct((NR, 128), jnp.float32),
            jax.ShapeDtypeStruct((NR, 128), jnp.float32),
        ),
    )(parts, q, b_row, wr_blk, wo_blk)


def _tcfin_body(parts_ref, q_ref, b_ref, o_ref):
    o_ref[...] = jnp.tanh(parts_ref[0] + parts_ref[1] + q_ref[...] + b_ref[...])


def _tcfin(parts, q, b_row):
    return pl.pallas_call(
        _tcfin_body,
        out_shape=jax.ShapeDtypeStruct((NR, 128), jnp.float32),
    )(parts, q, b_row)


@functools.partial(
    pl.kernel,
    out_type=jax.ShapeDtypeStruct((NCORES, NPF), jnp.float32),
    mesh=plsc.VectorSubcoreMesh(core_axis_name="c", subcore_axis_name="s"),
    scratch_types=[
        pltpu.VMEM_SHARED((NPF,), jnp.float32),
        pltpu.VMEM_SHARED((NPF,), jnp.float32),
        pltpu.VMEM((CS,), jnp.int32),
        pltpu.VMEM((CS,), jnp.int32),
        pltpu.VMEM((CS,), jnp.float32),
        pltpu.SemaphoreType.DMA,
    ],
)
def _sc_edge_agg(p_hbm, idx_hbm, z_hbm, out_hbm,
                 p_sp, agg_sp, sidx, didx, rows, sem):
    c = lax.axis_index("c")
    s = lax.axis_index("s")
    rp = NPF // NTILES
    sl = pl.ds(s * rp, rp)
    pltpu.sync_copy(p_hbm.at[sl], p_sp.at[sl])
    pltpu.sync_copy(z_hbm.at[sl], agg_sp.at[sl])
    pltpu.sync_copy(idx_hbm.at[c, s, 0, 0], sidx)
    pltpu.sync_copy(idx_hbm.at[c, s, 1, 0], didx)
    plsc.subcore_barrier()

    pltpu.async_copy(p_sp.at[sidx], rows, sem).wait()
    pltpu.sync_copy(rows, agg_sp.at[didx], add=True)

    plsc.subcore_barrier()
    pltpu.sync_copy(agg_sp.at[sl], out_hbm.at[c].at[sl])


def _expand(idx):
    e4 = (idx[:, None] * 4 + jnp.arange(4, dtype=jnp.int32)).reshape(-1)
    npad = EPF - 4 * E
    pad = (4 * N + (jnp.arange(npad, dtype=jnp.int32) % (NPF - 4 * N)))
    return jnp.concatenate([e4, pad]).reshape(NCORES, NTILES, CH, CS)


def _blk(w, pad_to=4):
    wp = jnp.pad(w, ((0, 4 - w.shape[0]), (0, 4 - w.shape[1])))
    return jnp.kron(jnp.eye(32, dtype=jnp.float32), wp)


def kernel(edge_index, x, W1_rel, b1_rel, W1_root,
           W2_rel, b2_rel, W2_root, W3_rel, b3_rel, W3_root):
    idx4 = jnp.stack([_expand(edge_index[0]), _expand(edge_index[1])], axis=2)
    x_r = jnp.pad(x, ((0, NP - N), (0, 0))).reshape(NR, 32 * D)
    zeros = jnp.zeros((NPF,), jnp.float32)

    w1r_blk = jnp.kron(jnp.eye(32, dtype=jnp.float32), W1_rel)
    w1o_blk = jnp.kron(jnp.eye(32, dtype=jnp.float32), W1_root)
    p1, q1 = _tc1(x_r, w1r_blk, w1o_blk)
    parts1 = _sc_edge_agg(p1.reshape(NPF), idx4, zeros)

    b1t = jnp.tile(b1_rel, 32).reshape(1, 128)
    p2, q2 = _tcmid(parts1.reshape(NCORES, NR, 128), q1, b1t,
                    _blk(W2_rel), _blk(W2_root))
    parts2 = _sc_edge_agg(p2.reshape(NPF), idx4, zeros)

    b2t = jnp.tile(b2_rel, 32).reshape(1, 128)
    p3, q3 = _tcmid(parts2.reshape(NCORES, NR, 128), q2, b2t,
                    _blk(W3_rel), _blk(W3_root))
    parts3 = _sc_edge_agg(p3.reshape(NPF), idx4, zeros)

    b3t = jnp.tile(jnp.pad(b3_rel, (0, 2)), 32).reshape(1, 128)
    out = _tcfin(parts3.reshape(NCORES, NR, 128), q3, b3t)
    return out.reshape(NP, 4)[:N, :2]


# speedup vs baseline: 6.7435x; 1.0002x over previous
import functools

import jax
import jax.numpy as jnp
from jax import lax
from jax.experimental import pallas as pl
from jax.experimental.pallas import tpu as pltpu
from jax.experimental.pallas import tpu_sc as plsc

N = 10000
E = 160000
D = 256
NP = 10240
NPF = NP * 4
NR = NP // 32
NTILES = 16
NCORES = 2
CS = 20480
CH = 1
EPF = NCORES * NTILES * CH * CS


def _tc1_body(x_ref, wr_ref, wo_ref, p_ref, q_ref):
    xb = x_ref[...]
    p_ref[...] = jnp.dot(xb, wr_ref[...], preferred_element_type=jnp.float32)
    q_ref[...] = jnp.dot(xb, wo_ref[...], preferred_element_type=jnp.float32)


def _tc1(x_r, wr_blk, wo_blk):
    return pl.pallas_call(
        _tc1_body,
        out_shape=(
            jax.ShapeDtypeStruct((NR, 128), jnp.float32),
            jax.ShapeDtypeStruct((NR, 128), jnp.float32),
        ),
    )(x_r, wr_blk, wo_blk)


def _tcmid_body(parts_ref, q_ref, b_ref, wr_ref, wo_ref, pn_ref, qn_ref):
    h = jnp.tanh(parts_ref[0] + parts_ref[1] + q_ref[...] + b_ref[...])
    pn_ref[...] = jnp.dot(h, wr_ref[...], preferred_element_type=jnp.float32)
    qn_ref[...] = jnp.dot(h, wo_ref[...], preferred_element_type=jnp.float32)


def _tcmid(parts, q, b_row, wr_blk, wo_blk):
    return pl.pallas_call(
        _tcmid_body,
        out_shape=(
            jax.ShapeDtypeStruct((NR, 128), jnp.float32),
            jax.ShapeDtypeStruct((NR, 128), jnp.float32),
        ),
    )(parts, q, b_row, wr_blk, wo_blk)


def _tcfin_body(parts_ref, q_ref, b_ref, o_ref):
    o_ref[...] = jnp.tanh(parts_ref[0] + parts_ref[1] + q_ref[...] + b_ref[...])


def _tcfin(parts, q, b_row):
    return pl.pallas_call(
        _tcfin_body,
        out_shape=jax.ShapeDtypeStruct((NR, 128), jnp.float32),
    )(parts, q, b_row)


@functools.partial(
    pl.kernel,
    out_type=jax.ShapeDtypeStruct((NCORES, NPF), jnp.float32),
    mesh=plsc.VectorSubcoreMesh(core_axis_name="c", subcore_axis_name="s"),
    scratch_types=[
        pltpu.VMEM_SHARED((NPF,), jnp.float32),
        pltpu.VMEM_SHARED((NPF,), jnp.float32),
        pltpu.VMEM((CS,), jnp.int32),
        pltpu.VMEM((CS,), jnp.int32),
        pltpu.VMEM((CS,), jnp.float32),
        pltpu.SemaphoreType.DMA,
    ],
)
def _sc_edge_agg(p_hbm, idx_hbm, z_hbm, out_hbm,
                 p_sp, agg_sp, sidx, didx, rows, sem):
    c = lax.axis_index("c")
    s = lax.axis_index("s")
    rp = NPF // NTILES
    sl = pl.ds(s * rp, rp)
    pltpu.sync_copy(p_hbm.at[sl], p_sp.at[sl])
    pltpu.sync_copy(z_hbm.at[sl], agg_sp.at[sl])
    pltpu.sync_copy(idx_hbm.at[c, s, 0, 0], sidx)
    pltpu.sync_copy(idx_hbm.at[c, s, 1, 0], didx)
    plsc.subcore_barrier()

    pltpu.async_copy(p_sp.at[sidx], rows, sem).wait()
    pltpu.sync_copy(rows, agg_sp.at[didx], add=True)

    plsc.subcore_barrier()
    pltpu.sync_copy(agg_sp.at[sl], out_hbm.at[c].at[sl])


def _expand(idx):
    e4 = (idx[:, None] * 4 + jnp.arange(4, dtype=jnp.int32)).reshape(-1)
    npad = EPF - 4 * E
    pad = (4 * N + (jnp.arange(npad, dtype=jnp.int32) % (NPF - 4 * N)))
    return jnp.concatenate([e4, pad]).reshape(NCORES, NTILES, CH, CS)


def _blk(w):
    wp = jnp.pad(w, ((0, 4 - w.shape[0]), (0, 4 - w.shape[1])))
    return jnp.kron(jnp.eye(32, dtype=jnp.float32), wp)


def kernel(edge_index, x, W1_rel, b1_rel, W1_root,
           W2_rel, b2_rel, W2_root, W3_rel, b3_rel, W3_root):
    idx4 = jnp.stack([_expand(edge_index[0]), _expand(edge_index[1])], axis=2)
    x_r = jnp.pad(x, ((0, NP - N), (0, 0))).reshape(NR, 32 * D)
    zeros = jnp.zeros((NPF,), jnp.float32)

    w1r_blk = jnp.kron(jnp.eye(32, dtype=jnp.float32), W1_rel)
    w1o_blk = jnp.kron(jnp.eye(32, dtype=jnp.float32), W1_root)
    p1, q1 = _tc1(x_r, w1r_blk, w1o_blk)
    parts1 = _sc_edge_agg(p1.reshape(NPF), idx4, zeros)

    b1t = jnp.tile(b1_rel, 32).reshape(1, 128)
    p2, q2 = _tcmid(parts1.reshape(NCORES, NR, 128), q1, b1t,
                    _blk(W2_rel), _blk(W2_root))
    parts2 = _sc_edge_agg(p2.reshape(NPF), idx4, zeros)

    b2t = jnp.tile(b2_rel, 32).reshape(1, 128)
    p3, q3 = _tcmid(parts2.reshape(NCORES, NR, 128), q2, b2t,
                    _blk(W3_rel), _blk(W3_root))
    parts3 = _sc_edge_agg(p3.reshape(NPF), idx4, zeros)

    b3t = jnp.tile(jnp.pad(b3_rel, (0, 2)), 32).reshape(1, 128)
    out = _tcfin(parts3.reshape(NCORES, NR, 128), q3, b3t)
    return out.reshape(NP, 4)[:N, :2]
